# baseline (speedup 1.0000x reference)
import functools

import jax
import jax.numpy as jnp
from jax import lax
from jax.experimental import pallas as pl
from jax.experimental.pallas import tpu as pltpu

_B, _H, _W = 4, 60, 80
_FLAT = _B * _H * _W
_FLATP = _FLAT + 8


def _pick_div(n, cands):
    for c in cands:
        if n % c == 0:
            return c
    return 1


def _fill_rows(ref, value):
    rows, cols = ref.shape
    blk = jnp.full((8, cols), value, jnp.float32)

    def body(i, carry):
        ref[pl.ds(pl.multiple_of(i * 8, 8), 8), :] = blk
        return carry

    lax.fori_loop(0, rows // 8, body, 0)


def _gather_row(ref, i):
    base = pl.multiple_of((i >> 3) << 3, 8)
    ch = ref[pl.ds(base, 8), :]
    return pltpu.roll(ch, -(i & 7), axis=0)[0:1, :]


def _prep_kernel(eb, src_ref, dst_ref, pos_ref, rel_ref):
    for mi in range(eb):
        ps = _gather_row(pos_ref, src_ref[0, mi])
        pd = _gather_row(pos_ref, dst_ref[0, mi])
        rel_ref[pl.ds(mi, 1), :] = ps - pd


def _dense_g_kernel(feat_ref, wf_ref, ba_ref, g_ref):
    g_ref[...] = jnp.dot(feat_ref[...], wf_ref[...],
                         preferred_element_type=jnp.float32) + ba_ref[...]


def _dense_g_acc_kernel(acc_ref, wf_ref, ba_ref, bbp_ref, g_ref):
    m = jnp.maximum(acc_ref[0], acc_ref[1])
    h = jnp.where(jnp.isfinite(m), m + bbp_ref[...], 0.0)
    g_ref[...] = jnp.dot(h, wf_ref[...],
                         preferred_element_type=jnp.float32) + ba_ref[...]


def _fixup_kernel(acc_ref, bbp_ref, h_ref):
    m = jnp.maximum(acc_ref[0], acc_ref[1])
    h_ref[...] = jnp.where(jnp.isfinite(m), m + bbp_ref[...], 0.0)


def _edge_kernel(eb, raw_wf, src_ref, dst_ref, g_ref, rel_ref, wr_ref, wb_ref,
                 *rest):
    if raw_wf:
        (wf_ref, ba_ref, acc_ref, tile_ref, msg_ref) = rest
    else:
        (acc_ref, tile_ref, msg_ref) = rest

    @pl.when(pl.program_id(1) == 0)
    def _():
        _fill_rows(acc_ref, -jnp.inf)

    for mi in range(eb):
        tile_ref[pl.ds(mi, 1), :] = _gather_row(g_ref, src_ref[0, mi])

    relw = jnp.dot(rel_ref[...], wr_ref[...], preferred_element_type=jnp.float32)
    if raw_wf:
        pre = jnp.dot(tile_ref[...], wf_ref[...],
                      preferred_element_type=jnp.float32) + relw + ba_ref[...]
    else:
        pre = tile_ref[...] + relw
    msg_ref[...] = jnp.dot(jnp.maximum(pre, 0.0), wb_ref[...],
                           preferred_element_type=jnp.float32)

    cout = msg_ref.shape[1]
    iota8 = lax.broadcasted_iota(jnp.int32, (8, cout), 0)
    for mi in range(eb):
        d = dst_ref[0, mi]
        db = pl.multiple_of((d >> 3) << 3, 8)
        ch = acc_ref[pl.ds(db, 8), :]
        mrow = msg_ref[pl.ds(mi, 1), :]
        sel = iota8 == (d & 7)
        acc_ref[pl.ds(db, 8), :] = jnp.where(sel, jnp.maximum(ch, mrow), ch)


def _image_kernel(np_, idx_ref, feat_ref, bb_ref, flat_ref):
    @pl.when(pl.program_id(0) == 0)
    def _():
        _fill_rows(flat_ref, 0.0)

    f = feat_ref[...]
    h = jnp.where(jnp.isfinite(f), f + bb_ref[...], 0.0)
    cout = h.shape[1]
    iota8 = lax.broadcasted_iota(jnp.int32, (8, cout), 0)
    for mi in range(np_):
        t = idx_ref[0, mi]
        tb = pl.multiple_of((t >> 3) << 3, 8)
        ch = flat_ref[pl.ds(tb, 8), :]
        sel = iota8 == (t & 7)
        flat_ref[pl.ds(tb, 8), :] = ch + jnp.where(sel, h[mi:mi + 1, :], 0.0)


def _full_spec(shape):
    nd = len(shape)
    return pl.BlockSpec(shape, lambda c, b, _n=nd: (0,) * nd)


_CPARAMS = pltpu.CompilerParams(
    dimension_semantics=("parallel", "arbitrary"),
    vmem_limit_bytes=56 * 1024 * 1024)


def _smem_idx_spec(eb, index_map):
    return pl.BlockSpec((None, 1, eb), index_map, memory_space=pltpu.SMEM)


def _prep_pass(pos8, src3, dst3, eb):
    nbt = dst3.shape[0]
    nbc = nbt // 2
    e = nbt * eb
    imap = lambda c, b: (c * nbc + b, 0, 0)
    return pl.pallas_call(
        functools.partial(_prep_kernel, eb),
        grid=(2, nbc),
        in_specs=[_smem_idx_spec(eb, imap), _smem_idx_spec(eb, imap),
                  _full_spec(pos8.shape)],
        out_specs=pl.BlockSpec((eb, 8), lambda c, b: (c * nbc + b, 0)),
        out_shape=jax.ShapeDtypeStruct((e, 8), jnp.float32),
        compiler_params=_CPARAMS,
    )(src3, dst3, pos8)


def _dense_pass(feat, wf, ba, prev_bias=None, fixup_only=False):
    n = feat.shape[0] if prev_bias is None else feat.shape[1]
    cm = feat.shape[2] if fixup_only else wf.shape[1]
    nb = _pick_div(n // 2, [1000, 600, 500, 200, 100, 40, 8, 1])
    nbn = n // 2 // nb
    if prev_bias is None:
        body = _dense_g_kernel
        specs = [pl.BlockSpec((nb, feat.shape[1]),
                              lambda c, b: (c * nbn + b, 0))]
    else:
        body = _fixup_kernel if fixup_only else _dense_g_acc_kernel
        specs = [pl.BlockSpec((2, nb, feat.shape[2]),
                              lambda c, b: (0, c * nbn + b, 0))]
    args = [feat]
    if not fixup_only:
        specs += [_full_spec(wf.shape), _full_spec(ba.shape)]
        args += [wf, ba]
    if prev_bias is not None:
        specs.append(_full_spec(prev_bias.shape))
        args.append(prev_bias)
    return pl.pallas_call(
        body,
        grid=(2, nbn),
        in_specs=specs,
        out_specs=pl.BlockSpec((nb, cm), lambda c, b: (c * nbn + b, 0)),
        out_shape=jax.ShapeDtypeStruct((n, cm), jnp.float32),
        compiler_params=_CPARAMS,
    )(*args)


def _edge_pass_split(g, rel, wr8, wb, src3, dst3, eb, nbc):
    n, cm = g.shape
    cout = wb.shape[1]
    imap = lambda c, b: (c * nbc + b, 0, 0)
    return pl.pallas_call(
        functools.partial(_edge_kernel, eb, False),
        grid=(2, nbc),
        in_specs=[
            _smem_idx_spec(eb, imap),
            _smem_idx_spec(eb, imap),
            _full_spec((n, cm)),
            pl.BlockSpec((eb, 8), lambda c, b: (c * nbc + b, 0)),
            _full_spec(wr8.shape),
            _full_spec(wb.shape),
        ],
        out_specs=pl.BlockSpec((None, n, cout), lambda c, b: (c, 0, 0)),
        out_shape=jax.ShapeDtypeStruct((2, n, cout), jnp.float32),
        scratch_shapes=[
            pltpu.VMEM((eb, cm), jnp.float32),
            pltpu.VMEM((eb, cout), jnp.float32),
        ],
        compiler_params=_CPARAMS,
    )(src3, dst3, g, rel, wr8, wb)


def _edge_pass_colsplit(h, rel, wf, ba, wr8, wb, src3, dst3, eb, nb3):
    n, cm = h.shape
    cout = wb.shape[1]
    chalf = cout // 2
    imap = lambda c, b: (b, 0, 0)
    return pl.pallas_call(
        functools.partial(_edge_kernel, eb, True),
        grid=(2, nb3),
        in_specs=[
            _smem_idx_spec(eb, imap),
            _smem_idx_spec(eb, imap),
            _full_spec((n, cm)),
            pl.BlockSpec((eb, 8), lambda c, b: (b, 0)),
            _full_spec(wr8.shape),
            pl.BlockSpec((wb.shape[0], chalf), lambda c, b: (0, c)),
            _full_spec(wf.shape),
            _full_spec(ba.shape),
        ],
        out_specs=pl.BlockSpec((n, chalf), lambda c, b: (0, c)),
        out_shape=jax.ShapeDtypeStruct((n, cout), jnp.float32),
        scratch_shapes=[
            pltpu.VMEM((eb, cm), jnp.float32),
            pltpu.VMEM((eb, chalf), jnp.float32),
        ],
        compiler_params=_CPARAMS,
    )(src3, dst3, h, rel, wr8, wb, wf, ba)


def _image_pass(feat3, idx3, bb3, np_):
    n, cout = feat3.shape
    nbp = n // np_
    return pl.pallas_call(
        functools.partial(_image_kernel, np_),
        grid=(nbp,),
        in_specs=[
            pl.BlockSpec((None, 1, np_), lambda b: (b, 0, 0),
                         memory_space=pltpu.SMEM),
            pl.BlockSpec((np_, cout), lambda b: (b, 0)),
            pl.BlockSpec(bb3.shape, lambda b: (0, 0)),
        ],
        out_specs=pl.BlockSpec((_FLATP, cout), lambda b: (0, 0)),
        out_shape=jax.ShapeDtypeStruct((_FLATP, cout), jnp.float32),
        compiler_params=pltpu.CompilerParams(
            dimension_semantics=("arbitrary",),
            vmem_limit_bytes=56 * 1024 * 1024),
    )(idx3, feat3, bb3)


def kernel(x, pos, K, w11, b11, w12, b12, w21, b21, w22, b22, w31, b31, w32,
           b32, edge_index, batch):
    n, cin = x.shape
    e = edge_index.shape[1]
    c1, c2, c3 = w12.shape[1], w22.shape[1], w32.shape[1]

    pos8 = jnp.pad(pos, ((0, 0), (0, 5)))
    eb = _pick_div(e // 2, [160, 100, 80, 50, 40, 32, 20, 16, 10, 8, 5, 4, 2, 1])
    src3 = edge_index[0].reshape(-1, 1, eb)
    dst3 = edge_index[1].reshape(-1, 1, eb)
    nbc = e // 2 // eb
    nb3 = e // eb

    rel = _prep_pass(pos8, src3, dst3, eb)

    wf1, wr1 = w11[:cin], jnp.pad(w11[cin:], ((0, 5), (0, 0)))
    g1 = _dense_pass(x, wf1, b11.reshape(1, -1))
    acc1 = _edge_pass_split(g1, rel, wr1, w12, src3, dst3, eb, nbc)

    wf2, wr2 = w21[:c1], jnp.pad(w21[c1:], ((0, 5), (0, 0)))
    g2 = _dense_pass(acc1, wf2, b21.reshape(1, -1), prev_bias=b12.reshape(1, -1))
    acc2 = _edge_pass_split(g2, rel, wr2, w22, src3, dst3, eb, nbc)

    wf3, wr3 = w31[:c2], jnp.pad(w31[c2:], ((0, 5), (0, 0)))
    h2 = _dense_pass(acc2, None, None, prev_bias=b22.reshape(1, -1),
                     fixup_only=True)
    feat3 = _edge_pass_colsplit(h2, rel, wf3, b31.reshape(1, -1), wr3, w32,
                                src3, dst3, eb, nb3)

    X, Y, Z = pos[:, 0], pos[:, 1], pos[:, 2]
    u = ((K[0, 0] * X / Z + K[0, 2]) / 640.0 * _W).astype(jnp.int32)
    v = ((K[1, 1] * Y / Z + K[1, 2]) / 480.0 * _H).astype(jnp.int32)
    mask = (u >= 0) & (u < _W) & (v >= 0) & (v < _H)
    idx = jnp.where(mask, batch * (_H * _W) + v * _W + u, _FLAT)
    np_ = _pick_div(n, [200, 100, 50, 40, 32, 16, 8, 4, 2, 1])
    idx3 = idx.astype(jnp.int32).reshape(-1, 1, np_)

    flat = _image_pass(feat3, idx3, b32.reshape(1, -1), np_)
    return flat[:_FLAT].reshape(_B, _H, _W, c3).transpose(0, 3, 1, 2)


# speedup vs baseline: 1.2252x; 1.2252x over previous
import functools

import jax
import jax.numpy as jnp
from jax import lax
from jax.experimental import pallas as pl
from jax.experimental.pallas import tpu as pltpu

_B, _H, _W = 4, 60, 80
_FLAT = _B * _H * _W
_FLATP = _FLAT + 8


def _pick_div(n, cands):
    for c in cands:
        if n % c == 0:
            return c
    return 1


def _fill_rows(ref, value):
    rows, cols = ref.shape
    blk = jnp.full((8, cols), value, jnp.float32)

    def body(i, carry):
        ref[pl.ds(pl.multiple_of(i * 8, 8), 8), :] = blk
        return carry

    lax.fori_loop(0, rows // 8, body, 0)


def _gather_row(ref, i):
    base = pl.multiple_of((i >> 3) << 3, 8)
    ch = ref[pl.ds(base, 8), :]
    return pltpu.roll(ch, -(i & 7), axis=0)[0:1, :]


def _prep_kernel(eb, src_ref, dst_ref, pos_ref, rel_ref):
    for mi in range(eb):
        ps = _gather_row(pos_ref, src_ref[0, mi])
        pd = _gather_row(pos_ref, dst_ref[0, mi])
        rel_ref[pl.ds(mi, 1), :] = ps - pd


def _dense_g_kernel(feat_ref, wf_ref, ba_ref, g_ref):
    g_ref[...] = jnp.dot(feat_ref[...], wf_ref[...],
                         preferred_element_type=jnp.float32) + ba_ref[...]


def _dense_g_acc_kernel(acc_ref, wf_ref, ba_ref, bbp_ref, g_ref):
    m = jnp.maximum(acc_ref[0], acc_ref[1])
    h = jnp.where(jnp.isfinite(m), m + bbp_ref[...], 0.0)
    g_ref[...] = jnp.dot(h, wf_ref[...],
                         preferred_element_type=jnp.float32) + ba_ref[...]


def _fixup_kernel(acc_ref, bbp_ref, h_ref):
    m = jnp.maximum(acc_ref[0], acc_ref[1])
    h_ref[...] = jnp.where(jnp.isfinite(m), m + bbp_ref[...], 0.0)


def _edge_kernel(eb, raw_wf, src_ref, dst_ref, g_ref, rel_ref, wr_ref, wb_ref,
                 *rest):
    if raw_wf:
        (wf_ref, ba_ref, acc_ref, tile_ref, msg_ref) = rest
    else:
        (acc_ref, tile_ref, msg_ref) = rest

    @pl.when(pl.program_id(1) == 0)
    def _():
        _fill_rows(acc_ref, -jnp.inf)

    for mi in range(eb):
        tile_ref[pl.ds(mi, 1), :] = _gather_row(g_ref, src_ref[0, mi])

    relw = jnp.dot(rel_ref[...], wr_ref[...], preferred_element_type=jnp.float32)
    if raw_wf:
        pre = jnp.dot(tile_ref[...], wf_ref[...],
                      preferred_element_type=jnp.float32) + relw + ba_ref[...]
    else:
        pre = tile_ref[...] + relw
    msg_ref[...] = jnp.dot(jnp.maximum(pre, 0.0), wb_ref[...],
                           preferred_element_type=jnp.float32)

    cout = msg_ref.shape[1]
    iota8 = lax.broadcasted_iota(jnp.int32, (8, cout), 0)
    for mi in range(eb):
        d = dst_ref[0, mi]
        db = pl.multiple_of((d >> 3) << 3, 8)
        ch = acc_ref[pl.ds(db, 8), :]
        mrow = msg_ref[pl.ds(mi, 1), :]
        sel = iota8 == (d & 7)
        acc_ref[pl.ds(db, 8), :] = jnp.where(sel, jnp.maximum(ch, mrow), ch)


def _edge3_kernel(eb, nsteps, src_ref, dst_ref, g_ref, rel_ref, wr_ref,
                  wb_ref, wf_ref, ba_ref, out_ref, acc_ref, tile_ref, msg_ref,
                  sem):
    @pl.when(pl.program_id(1) == 0)
    def _():
        _fill_rows(acc_ref, -jnp.inf)

    for mi in range(eb):
        tile_ref[pl.ds(mi, 1), :] = _gather_row(g_ref, src_ref[0, mi])

    relw = jnp.dot(rel_ref[...], wr_ref[...], preferred_element_type=jnp.float32)
    pre = jnp.dot(tile_ref[...], wf_ref[...],
                  preferred_element_type=jnp.float32) + relw + ba_ref[...]
    msg_ref[...] = jnp.dot(jnp.maximum(pre, 0.0), wb_ref[...],
                           preferred_element_type=jnp.float32)

    cout = msg_ref.shape[1]
    iota8 = lax.broadcasted_iota(jnp.int32, (8, cout), 0)
    for mi in range(eb):
        d = dst_ref[0, mi]
        db = pl.multiple_of((d >> 3) << 3, 8)
        ch = acc_ref[pl.ds(db, 8), :]
        mrow = msg_ref[pl.ds(mi, 1), :]
        sel = iota8 == (d & 7)
        acc_ref[pl.ds(db, 8), :] = jnp.where(sel, jnp.maximum(ch, mrow), ch)

    @pl.when(pl.program_id(1) == nsteps - 1)
    def _():
        cp = pltpu.make_async_copy(acc_ref, out_ref.at[pl.program_id(0)], sem)
        cp.start()
        cp.wait()


def _image_kernel(np_, idx_ref, feat_ref, bb_ref, flat_ref):
    @pl.when(pl.program_id(0) == 0)
    def _():
        _fill_rows(flat_ref, 0.0)

    f = jnp.maximum(feat_ref[0], feat_ref[1])
    h = jnp.where(jnp.isfinite(f), f + bb_ref[...], 0.0)
    cout = h.shape[1]
    iota8 = lax.broadcasted_iota(jnp.int32, (8, cout), 0)
    for mi in range(np_):
        t = idx_ref[0, mi]
        tb = pl.multiple_of((t >> 3) << 3, 8)
        ch = flat_ref[pl.ds(tb, 8), :]
        sel = iota8 == (t & 7)
        flat_ref[pl.ds(tb, 8), :] = ch + jnp.where(sel, h[mi:mi + 1, :], 0.0)


def _full_spec(shape):
    nd = len(shape)
    return pl.BlockSpec(shape, lambda c, b, _n=nd: (0,) * nd)


_CPARAMS = pltpu.CompilerParams(
    dimension_semantics=("parallel", "arbitrary"),
    vmem_limit_bytes=56 * 1024 * 1024)


def _smem_idx_spec(eb, index_map):
    return pl.BlockSpec((None, 1, eb), index_map, memory_space=pltpu.SMEM)


def _prep_pass(pos8, src3, dst3, eb):
    nbt = dst3.shape[0]
    nbc = nbt // 2
    e = nbt * eb
    imap = lambda c, b: (c * nbc + b, 0, 0)
    return pl.pallas_call(
        functools.partial(_prep_kernel, eb),
        grid=(2, nbc),
        in_specs=[_smem_idx_spec(eb, imap), _smem_idx_spec(eb, imap),
                  _full_spec(pos8.shape)],
        out_specs=pl.BlockSpec((eb, 8), lambda c, b: (c * nbc + b, 0)),
        out_shape=jax.ShapeDtypeStruct((e, 8), jnp.float32),
        compiler_params=_CPARAMS,
    )(src3, dst3, pos8)


def _dense_pass(feat, wf, ba, prev_bias=None, fixup_only=False):
    n = feat.shape[0] if prev_bias is None else feat.shape[1]
    cm = feat.shape[2] if fixup_only else wf.shape[1]
    nb = _pick_div(n // 2, [1000, 600, 500, 200, 100, 40, 8, 1])
    nbn = n // 2 // nb
    if prev_bias is None:
        body = _dense_g_kernel
        specs = [pl.BlockSpec((nb, feat.shape[1]),
                              lambda c, b: (c * nbn + b, 0))]
    else:
        body = _fixup_kernel if fixup_only else _dense_g_acc_kernel
        specs = [pl.BlockSpec((2, nb, feat.shape[2]),
                              lambda c, b: (0, c * nbn + b, 0))]
    args = [feat]
    if not fixup_only:
        specs += [_full_spec(wf.shape), _full_spec(ba.shape)]
        args += [wf, ba]
    if prev_bias is not None:
        specs.append(_full_spec(prev_bias.shape))
        args.append(prev_bias)
    return pl.pallas_call(
        body,
        grid=(2, nbn),
        in_specs=specs,
        out_specs=pl.BlockSpec((nb, cm), lambda c, b: (c * nbn + b, 0)),
        out_shape=jax.ShapeDtypeStruct((n, cm), jnp.float32),
        compiler_params=_CPARAMS,
    )(*args)


def _edge_pass_split(g, rel, wr8, wb, src3, dst3, eb, nbc):
    n, cm = g.shape
    cout = wb.shape[1]
    imap = lambda c, b: (c * nbc + b, 0, 0)
    return pl.pallas_call(
        functools.partial(_edge_kernel, eb, False),
        grid=(2, nbc),
        in_specs=[
            _smem_idx_spec(eb, imap),
            _smem_idx_spec(eb, imap),
            _full_spec((n, cm)),
            pl.BlockSpec((eb, 8), lambda c, b: (c * nbc + b, 0)),
            _full_spec(wr8.shape),
            _full_spec(wb.shape),
        ],
        out_specs=pl.BlockSpec((None, n, cout), lambda c, b: (c, 0, 0)),
        out_shape=jax.ShapeDtypeStruct((2, n, cout), jnp.float32),
        scratch_shapes=[
            pltpu.VMEM((eb, cm), jnp.float32),
            pltpu.VMEM((eb, cout), jnp.float32),
        ],
        compiler_params=_CPARAMS,
    )(src3, dst3, g, rel, wr8, wb)


def _edge_pass3(h, rel, wf, ba, wr8, wb, src3, dst3, eb, nbc):
    n, cm = h.shape
    cout = wb.shape[1]
    imap = lambda c, b: (c * nbc + b, 0, 0)
    return pl.pallas_call(
        functools.partial(_edge3_kernel, eb, nbc),
        grid=(2, nbc),
        in_specs=[
            _smem_idx_spec(eb, imap),
            _smem_idx_spec(eb, imap),
            _full_spec((n, cm)),
            pl.BlockSpec((eb, 8), lambda c, b: (c * nbc + b, 0)),
            _full_spec(wr8.shape),
            _full_spec(wb.shape),
            _full_spec(wf.shape),
            _full_spec(ba.shape),
        ],
        out_specs=pl.BlockSpec(memory_space=pl.ANY),
        out_shape=jax.ShapeDtypeStruct((2, n, cout), jnp.float32),
        scratch_shapes=[
            pltpu.VMEM((n, cout), jnp.float32),
            pltpu.VMEM((eb, cm), jnp.float32),
            pltpu.VMEM((eb, cout), jnp.float32),
            pltpu.SemaphoreType.DMA,
        ],
        compiler_params=_CPARAMS,
    )(src3, dst3, h, rel, wr8, wb, wf, ba)


def _image_pass(feat3, idx3, bb3, np_):
    _, n, cout = feat3.shape
    nbp = n // np_
    return pl.pallas_call(
        functools.partial(_image_kernel, np_),
        grid=(nbp,),
        in_specs=[
            pl.BlockSpec((None, 1, np_), lambda b: (b, 0, 0),
                         memory_space=pltpu.SMEM),
            pl.BlockSpec((2, np_, cout), lambda b: (0, b, 0)),
            pl.BlockSpec(bb3.shape, lambda b: (0, 0)),
        ],
        out_specs=pl.BlockSpec((_FLATP, cout), lambda b: (0, 0)),
        out_shape=jax.ShapeDtypeStruct((_FLATP, cout), jnp.float32),
        compiler_params=pltpu.CompilerParams(
            dimension_semantics=("arbitrary",),
            vmem_limit_bytes=56 * 1024 * 1024),
    )(idx3, feat3, bb3)


def kernel(x, pos, K, w11, b11, w12, b12, w21, b21, w22, b22, w31, b31, w32,
           b32, edge_index, batch):
    n, cin = x.shape
    e = edge_index.shape[1]
    c1, c2, c3 = w12.shape[1], w22.shape[1], w32.shape[1]

    pos8 = jnp.pad(pos, ((0, 0), (0, 5)))
    eb = _pick_div(e // 2, [160, 100, 80, 50, 40, 32, 20, 16, 10, 8, 5, 4, 2, 1])
    src3 = edge_index[0].reshape(-1, 1, eb)
    dst3 = edge_index[1].reshape(-1, 1, eb)
    nbc = e // 2 // eb
    nb3 = e // eb

    rel = _prep_pass(pos8, src3, dst3, eb)

    wf1, wr1 = w11[:cin], jnp.pad(w11[cin:], ((0, 5), (0, 0)))
    g1 = _dense_pass(x, wf1, b11.reshape(1, -1))
    acc1 = _edge_pass_split(g1, rel, wr1, w12, src3, dst3, eb, nbc)

    wf2, wr2 = w21[:c1], jnp.pad(w21[c1:], ((0, 5), (0, 0)))
    g2 = _dense_pass(acc1, wf2, b21.reshape(1, -1), prev_bias=b12.reshape(1, -1))
    acc2 = _edge_pass_split(g2, rel, wr2, w22, src3, dst3, eb, nbc)

    wf3, wr3 = w31[:c2], jnp.pad(w31[c2:], ((0, 5), (0, 0)))
    h2 = _dense_pass(acc2, None, None, prev_bias=b22.reshape(1, -1),
                     fixup_only=True)
    feat3 = _edge_pass3(h2, rel, wf3, b31.reshape(1, -1), wr3, w32,
                        src3, dst3, eb, nbc)

    X, Y, Z = pos[:, 0], pos[:, 1], pos[:, 2]
    u = ((K[0, 0] * X / Z + K[0, 2]) / 640.0 * _W).astype(jnp.int32)
    v = ((K[1, 1] * Y / Z + K[1, 2]) / 480.0 * _H).astype(jnp.int32)
    mask = (u >= 0) & (u < _W) & (v >= 0) & (v < _H)
    idx = jnp.where(mask, batch * (_H * _W) + v * _W + u, _FLAT)
    np_ = _pick_div(n, [200, 100, 50, 40, 32, 16, 8, 4, 2, 1])
    idx3 = idx.astype(jnp.int32).reshape(-1, 1, np_)

    flat = _image_pass(feat3, idx3, b32.reshape(1, -1), np_)
    return flat[:_FLAT].reshape(_B, _H, _W, c3).transpose(0, 3, 1, 2)
